# baseline (speedup 1.0000x reference)
import jax
import jax.numpy as jnp
from jax import lax
from jax.experimental import pallas as pl
from jax.experimental.pallas import tpu as pltpu
from jax.experimental.pallas import tpu_sc as plsc

B = 8
HW = 112 * 112
P = 224 * 224
C = 96
NCB = 6
CB = 16
NC = 2
NS = 16
ROWS = HW // NS
PROWS = P // NS
ZW = PROWS
TASKS_PER_CORE = (B * NCB) // NC
UNROLL = 4


def _task_coords(cid, t):
  task = cid * TASKS_PER_CORE + t
  return task // NCB, (task % NCB) * CB


def _body(net_ref, mask_ref, out_ref,
          accum, mask_v, vals2_v, vals_v, idx_v, zero_v, drain_f, drain2,
          sem_z, sem_in, sem_out):
  cid = lax.axis_index("c")
  sid = lax.axis_index("s")
  lane = lax.iota(jnp.int32, 16)
  zf16 = jnp.zeros((16,), jnp.float32)

  def _zinit(j, _):
    zero_v[pl.ds(j * 16, 16)] = zf16
    return 0
  lax.fori_loop(0, ZW // 16, _zinit, 0)

  r0 = sid * ROWS

  def _stage_descs(b, cb):
    return (
        pltpu.make_async_copy(
            mask_ref.at[b, pl.ds(r0, ROWS), pl.ds(cb, CB)], mask_v, sem_in),
        pltpu.make_async_copy(
            net_ref.at[b, pl.ds(r0, ROWS), pl.ds(cb, CB)], vals2_v, sem_in),
    )

  b0, cb0 = _task_coords(cid, 0)
  for d in _stage_descs(b0, cb0):
    d.start()

  def task_body(t, _):
    b, cb = _task_coords(cid, t)
    base = sid * PROWS * CB

    zdescs = [
        pltpu.make_async_copy(
            zero_v, accum.at[pl.ds(base + q * ZW, ZW)], sem_z)
        for q in range(PROWS * CB // ZW)
    ]
    for d in zdescs:
      d.start()

    for d in _stage_descs(b, cb):
      d.wait()

    def decode(j4, _):
      for u in range(UNROLL):
        j = j4 * UNROLL + u
        m = mask_v[j]
        x = m >> 5
        a = x >> 10
        r = x & 1023
        p = a * 341 + (((a + r) * 683) >> 11)
        idx_v[pl.ds(j * 16, 16)] = p * CB + lane
        vals_v[pl.ds(j * 16, 16)] = vals2_v[j]
      return 0
    lax.fori_loop(0, ROWS // UNROLL, decode, 0)

    for d in zdescs:
      d.wait()

    plsc.subcore_barrier()

    pltpu.sync_copy(vals_v, accum.at[idx_v], add=True)

    plsc.subcore_barrier()

    @pl.when(t + 1 < TASKS_PER_CORE)
    def _():
      bn, cbn = _task_coords(cid, t + 1)
      for d in _stage_descs(bn, cbn):
        d.start()

    outd = None
    for q in range(4):
      pltpu.sync_copy(accum.at[pl.ds(base + q * HW, HW)], drain_f)
      if outd is not None:
        outd.wait()

      def review(j4, _):
        for u in range(UNROLL):
          j = j4 * UNROLL + u
          drain2[j] = drain_f[pl.ds(j * 16, 16)]
        return 0
      lax.fori_loop(0, ROWS // UNROLL, review, 0)

      outd = pltpu.make_async_copy(
          drain2,
          out_ref.at[b, pl.ds(sid * PROWS + q * ROWS, ROWS), pl.ds(cb, CB)],
          sem_out)
      outd.start()
    outd.wait()
    return 0

  lax.fori_loop(0, TASKS_PER_CORE, task_body, 0)


@jax.jit
def kernel(net, mask):
  net3 = net.reshape(B, HW, C)
  mask3 = mask.reshape(B, HW, C)
  mesh = plsc.VectorSubcoreMesh(
      core_axis_name="c", subcore_axis_name="s", num_cores=NC, num_subcores=NS)
  f = pl.kernel(
      _body,
      out_type=jax.ShapeDtypeStruct((B, P, C), jnp.float32),
      mesh=mesh,
      compiler_params=pltpu.CompilerParams(use_tc_tiling_on_sc=False),
      scratch_types=[
          pltpu.VMEM_SHARED((P * CB,), jnp.float32),
          pltpu.VMEM((ROWS, CB), jnp.int32),
          pltpu.VMEM((ROWS, CB), jnp.float32),
          pltpu.VMEM((HW,), jnp.float32),
          pltpu.VMEM((HW,), jnp.int32),
          pltpu.VMEM((ZW,), jnp.float32),
          pltpu.VMEM((HW,), jnp.float32),
          pltpu.VMEM((ROWS, CB), jnp.float32),
          pltpu.SemaphoreType.DMA,
          pltpu.SemaphoreType.DMA,
          pltpu.SemaphoreType.DMA,
      ],
  )
  out = f(net3, mask3)
  return out.reshape(B, 224, 224, C)


# speedup vs baseline: 1.0283x; 1.0283x over previous
import jax
import jax.numpy as jnp
from jax import lax
from jax.experimental import pallas as pl
from jax.experimental.pallas import tpu as pltpu
from jax.experimental.pallas import tpu_sc as plsc

B = 8
HW = 112 * 112
P = 224 * 224
C = 96
NCB = 6
CB = 16
NC = 2
NS = 16
ROWS = HW // NS
PROWS = P // NS
ZW = PROWS
TASKS_PER_CORE = (B * NCB) // NC
UNROLL = 8


def _task_coords(cid, t):
  task = cid * TASKS_PER_CORE + t
  return task // NCB, (task % NCB) * CB


def _body(net_ref, mask_ref, out_ref,
          accum, mask_v, vals2_v, vals_v, idx_v, zero_v, drain_f, drain2,
          sem_z, sem_in, sem_out):
  cid = lax.axis_index("c")
  sid = lax.axis_index("s")
  lane = lax.iota(jnp.int32, 16)
  zf16 = jnp.zeros((16,), jnp.float32)

  def _zinit(j, _):
    zero_v[pl.ds(j * 16, 16)] = zf16
    return 0
  lax.fori_loop(0, ZW // 16, _zinit, 0)

  r0 = sid * ROWS

  def _stage_descs(b, cb):
    return (
        pltpu.make_async_copy(
            mask_ref.at[b, pl.ds(r0, ROWS), pl.ds(cb, CB)], mask_v, sem_in),
        pltpu.make_async_copy(
            net_ref.at[b, pl.ds(r0, ROWS), pl.ds(cb, CB)], vals2_v, sem_in),
    )

  b0, cb0 = _task_coords(cid, 0)
  for d in _stage_descs(b0, cb0):
    d.start()

  def task_body(t, _):
    b, cb = _task_coords(cid, t)
    base = sid * PROWS * CB

    zdescs = [
        pltpu.make_async_copy(
            zero_v, accum.at[pl.ds(base + q * ZW, ZW)], sem_z)
        for q in range(PROWS * CB // ZW)
    ]
    for d in zdescs:
      d.start()

    for d in _stage_descs(b, cb):
      d.wait()

    def decode(j4, _):
      for u in range(UNROLL):
        j = j4 * UNROLL + u
        m = mask_v[j]
        x = m >> 5
        a = x >> 10
        r = x & 1023
        p = a * 341 + (((a + r) * 683) >> 11)
        idx_v[pl.ds(j * 16, 16)] = p * CB + lane
        vals_v[pl.ds(j * 16, 16)] = vals2_v[j]
      return 0
    lax.fori_loop(0, ROWS // UNROLL, decode, 0)

    for d in zdescs:
      d.wait()

    plsc.subcore_barrier()

    pltpu.sync_copy(vals_v, accum.at[idx_v], add=True)

    plsc.subcore_barrier()

    @pl.when(t + 1 < TASKS_PER_CORE)
    def _():
      bn, cbn = _task_coords(cid, t + 1)
      for d in _stage_descs(bn, cbn):
        d.start()

    outd = None
    for q in range(4):
      pltpu.sync_copy(accum.at[pl.ds(base + q * HW, HW)], drain_f)
      if outd is not None:
        outd.wait()

      def review(j4, _):
        for u in range(UNROLL):
          j = j4 * UNROLL + u
          drain2[j] = drain_f[pl.ds(j * 16, 16)]
        return 0
      lax.fori_loop(0, ROWS // UNROLL, review, 0)

      outd = pltpu.make_async_copy(
          drain2,
          out_ref.at[b, pl.ds(sid * PROWS + q * ROWS, ROWS), pl.ds(cb, CB)],
          sem_out)
      outd.start()
    outd.wait()
    return 0

  lax.fori_loop(0, TASKS_PER_CORE, task_body, 0)


@jax.jit
def kernel(net, mask):
  net3 = net.reshape(B, HW, C)
  mask3 = mask.reshape(B, HW, C)
  mesh = plsc.VectorSubcoreMesh(
      core_axis_name="c", subcore_axis_name="s", num_cores=NC, num_subcores=NS)
  f = pl.kernel(
      _body,
      out_type=jax.ShapeDtypeStruct((B, P, C), jnp.float32),
      mesh=mesh,
      compiler_params=pltpu.CompilerParams(use_tc_tiling_on_sc=False),
      scratch_types=[
          pltpu.VMEM_SHARED((P * CB,), jnp.float32),
          pltpu.VMEM((ROWS, CB), jnp.int32),
          pltpu.VMEM((ROWS, CB), jnp.float32),
          pltpu.VMEM((HW,), jnp.float32),
          pltpu.VMEM((HW,), jnp.int32),
          pltpu.VMEM((ZW,), jnp.float32),
          pltpu.VMEM((HW,), jnp.float32),
          pltpu.VMEM((ROWS, CB), jnp.float32),
          pltpu.SemaphoreType.DMA,
          pltpu.SemaphoreType.DMA,
          pltpu.SemaphoreType.DMA,
      ],
  )
  out = f(net3, mask3)
  return out.reshape(B, 224, 224, C)
